# baseline (speedup 1.0000x reference)
import jax
import jax.numpy as jnp
from jax import lax
from jax.experimental import pallas as pl
from jax.experimental.pallas import tpu as pltpu
from jax.experimental.pallas import tpu_sc as plsc

N = 10000
E = 320000
D = 128
NEG_SLOPE = 0.01
EPS = 1e-5

NUM_CORES = 2
NUM_SUBCORES = 16
NUM_TILES = NUM_CORES * NUM_SUBCORES
N_PAD = 10240
ROWS_PER_TILE = N_PAD // NUM_TILES

CHUNK = 10000
NCHUNK = E // CHUNK
VPC = 16
CHECK_VREGS = 25
CHECKS_PER_CHUNK = CHUNK // (CHECK_VREGS * VPC)
FIRE = 80
PCAP = 640


def _leaky(v):
    return jnp.where(v >= 0, v, NEG_SLOPE * v)


def _sc_agg_body(y_hbm, src_hbm, dst_hbm, ea_hbm, out_hbm,
                 srcc0, dstc0, eac0, srcc1, dstc1, eac1,
                 pend, psrc, pdl, pea, rows_v, acc, sem0, sem1, gsem):
    c = lax.axis_index("c")
    s = lax.axis_index("s")
    tid = c * NUM_SUBCORES + s
    tid_vec = jnp.full((VPC,), tid, jnp.int32)
    iota = lax.iota(jnp.int32, VPC)
    zeros16 = jnp.zeros((VPC,), jnp.float32)
    zeros16i = jnp.zeros((VPC,), jnp.int32)

    def zrow(r, carry):
        for i in range(D // VPC):
            acc[r, pl.ds(i * VPC, VPC)] = zeros16
        return carry
    lax.fori_loop(0, ROWS_PER_TILE, zrow, 0)
    for t in range(FIRE // VPC):
        psrc[pl.ds(t * VPC, VPC)] = zeros16i

    def prefetch(g, srcb, dstb, eab, sem):
        o = g * CHUNK
        pltpu.async_copy(src_hbm.at[pl.ds(o, CHUNK)], srcb.at[pl.ds(0, CHUNK)], sem)
        pltpu.async_copy(dst_hbm.at[pl.ds(o, CHUNK)], dstb.at[pl.ds(0, CHUNK)], sem)
        pltpu.async_copy(ea_hbm.at[pl.ds(o, CHUNK)], eab.at[pl.ds(0, CHUNK)], sem)

    def wait_prefetch(g, srcb, dstb, eab, sem):
        o = g * CHUNK
        pltpu.make_async_copy(src_hbm.at[pl.ds(o, CHUNK)], srcb.at[pl.ds(0, CHUNK)], sem).wait()
        pltpu.make_async_copy(dst_hbm.at[pl.ds(o, CHUNK)], dstb.at[pl.ds(0, CHUNK)], sem).wait()
        pltpu.make_async_copy(ea_hbm.at[pl.ds(o, CHUNK)], eab.at[pl.ds(0, CHUNK)], sem).wait()

    def make_scan_drain(srcb, dstb, eab):
        def append_vreg(base, np_vec):
            dvec = dstb[pl.ds(base, VPC)]
            owner = lax.shift_right_logical(
                lax.shift_right_logical(dvec, 6) * 52429, 18)
            m = owner == tid_vec
            m_i = jnp.where(m, 1, 0).astype(jnp.int32)
            pos = plsc.cumsum(m_i) - m_i
            cnt = plsc.all_reduce_population_count(m)
            plsc.store_scatter(pend, [np_vec + pos], iota + base, mask=m)
            return np_vec + cnt

        def fire(f, np_s):
            off = pl.multiple_of(f * FIRE, FIRE)
            cnt_v = zeros16i + (np_s - off)
            for t in range(FIRE // VPC):
                o = pl.multiple_of(off + t * VPC, VPC)
                valid = (iota + (t * VPC)) < cnt_v
                p16 = jnp.where(valid, pend[pl.ds(o, VPC)], 0)
                sv = plsc.load_gather(srcb, [p16])
                av = plsc.load_gather(eab, [p16])
                dv = plsc.load_gather(dstb, [p16])
                av = jnp.where(valid, av, 0.0)
                owner = lax.shift_right_logical(
                    lax.shift_right_logical(dv, 6) * 52429, 18)
                psrc[pl.ds(t * VPC, VPC)] = sv
                pea[pl.ds(t * VPC, VPC)] = av
                pdl[pl.ds(t * VPC, VPC)] = dv - owner * ROWS_PER_TILE
            pltpu.async_copy(y_hbm.at[psrc.at[pl.ds(0, FIRE)]], rows_v, gsem).wait()

            for k in range(FIRE):
                pidx = zeros16i + k
                ea_b = plsc.load_gather(pea, [pidx])
                dl_b = plsc.load_gather(pdl, [pidx])
                for i in range(D // VPC):
                    v = rows_v[k, pl.ds(i * VPC, VPC)] * ea_b
                    plsc.addupdate_scatter(acc, [dl_b, iota + (i * VPC)], v)
            return np_s

        def scan_drain(np_vec):
            def app(u, np_vec3):
                for uu in range(5):
                    np_vec3 = append_vreg((u * 5 + uu) * VPC, np_vec3)
                return np_vec3
            np_vec = lax.fori_loop(0, CHUNK // (5 * VPC), app, np_vec)
            np_s = jnp.max(np_vec)
            nf = (np_s + FIRE - 1) // FIRE
            lax.fori_loop(0, nf, fire, np_s)
            return zeros16i
        return scan_drain

    scan_drain0 = make_scan_drain(srcc0, dstc0, eac0)
    scan_drain1 = make_scan_drain(srcc1, dstc1, eac1)

    prefetch(0, srcc0, dstc0, eac0, sem0)

    def pair(h, np_vec):
        g0 = h * 2
        wait_prefetch(g0, srcc0, dstc0, eac0, sem0)
        prefetch(g0 + 1, srcc1, dstc1, eac1, sem1)
        np_vec = scan_drain0(np_vec)
        wait_prefetch(g0 + 1, srcc1, dstc1, eac1, sem1)

        @pl.when(h < NCHUNK // 2 - 1)
        def _():
            prefetch(g0 + 2, srcc0, dstc0, eac0, sem0)
        return scan_drain1(np_vec)

    lax.fori_loop(0, NCHUNK // 2, pair, zeros16i)

    pltpu.sync_copy(acc, out_hbm.at[pl.ds(tid * ROWS_PER_TILE, ROWS_PER_TILE)])


_sc_agg = pl.kernel(
    _sc_agg_body,
    out_type=jax.ShapeDtypeStruct((N_PAD, D), jnp.float32),
    mesh=plsc.VectorSubcoreMesh(core_axis_name="c", subcore_axis_name="s",
                                num_cores=NUM_CORES,
                                num_subcores=NUM_SUBCORES),
    compiler_params=pltpu.CompilerParams(needs_layout_passes=False),
    scratch_types=[
        pltpu.VMEM((CHUNK,), jnp.int32),
        pltpu.VMEM((CHUNK,), jnp.int32),
        pltpu.VMEM((CHUNK,), jnp.float32),
        pltpu.VMEM((CHUNK,), jnp.int32),
        pltpu.VMEM((CHUNK,), jnp.int32),
        pltpu.VMEM((CHUNK,), jnp.float32),
        pltpu.VMEM((CHUNK + VPC,), jnp.int32),
        pltpu.VMEM((FIRE,), jnp.int32),
        pltpu.VMEM((FIRE,), jnp.int32),
        pltpu.VMEM((FIRE,), jnp.float32),
        pltpu.VMEM((FIRE, D), jnp.float32),
        pltpu.VMEM((ROWS_PER_TILE, D), jnp.float32),
        pltpu.SemaphoreType.DMA,
        pltpu.SemaphoreType.DMA,
        pltpu.SemaphoreType.DMA,
    ],
)


def _post_body(a_ref, x_ref, wr_ref, wt_ref, br_ref, wl_ref, bl_ref,
               x3_ref, s_ref, q_ref):
    i = pl.program_id(0)
    x1 = (jnp.dot(a_ref[...], wr_ref[...], preferred_element_type=jnp.float32)
          + jnp.dot(x_ref[...], wt_ref[...], preferred_element_type=jnp.float32)
          + br_ref[...])
    x2 = _leaky(x1)
    x3 = jnp.dot(x2, wl_ref[...], preferred_element_type=jnp.float32) + bl_ref[...]
    x3_ref[...] = x3

    @pl.when(i == 0)
    def _():
        s_ref[...] = jnp.zeros_like(s_ref)
        q_ref[...] = jnp.zeros_like(q_ref)

    s_ref[...] += jnp.sum(x3, axis=0, keepdims=True)
    q_ref[...] += jnp.sum(x3 * x3, axis=0, keepdims=True)


def _post(a, x, wrT, wtT, br, wlT, bl):
    blk = 1000
    grid = N // blk
    return pl.pallas_call(
        _post_body,
        grid=(grid,),
        in_specs=[
            pl.BlockSpec((blk, D), lambda i: (i, 0)),
            pl.BlockSpec((blk, D), lambda i: (i, 0)),
            pl.BlockSpec((D, D), lambda i: (0, 0)),
            pl.BlockSpec((D, D), lambda i: (0, 0)),
            pl.BlockSpec((1, D), lambda i: (0, 0)),
            pl.BlockSpec((D, D), lambda i: (0, 0)),
            pl.BlockSpec((1, D), lambda i: (0, 0)),
        ],
        out_specs=[
            pl.BlockSpec((blk, D), lambda i: (i, 0)),
            pl.BlockSpec((1, D), lambda i: (0, 0)),
            pl.BlockSpec((1, D), lambda i: (0, 0)),
        ],
        out_shape=[
            jax.ShapeDtypeStruct((N, D), jnp.float32),
            jax.ShapeDtypeStruct((1, D), jnp.float32),
            jax.ShapeDtypeStruct((1, D), jnp.float32),
        ],
    )(a, x, wrT, wtT, br, wlT, bl)


def _bn_body(x3_ref, s_ref, q_ref, g_ref, b_ref, o_ref):
    mean = s_ref[...] / N
    var = q_ref[...] / N - mean * mean
    scale = lax.rsqrt(var + EPS) * g_ref[...]
    x4 = (x3_ref[...] - mean) * scale + b_ref[...]
    o_ref[...] = _leaky(x4)


def _bn(x3, s, q, g, b):
    blk = 1000
    grid = N // blk
    return pl.pallas_call(
        _bn_body,
        grid=(grid,),
        in_specs=[
            pl.BlockSpec((blk, D), lambda i: (i, 0)),
            pl.BlockSpec((1, D), lambda i: (0, 0)),
            pl.BlockSpec((1, D), lambda i: (0, 0)),
            pl.BlockSpec((1, D), lambda i: (0, 0)),
            pl.BlockSpec((1, D), lambda i: (0, 0)),
        ],
        out_specs=pl.BlockSpec((blk, D), lambda i: (i, 0)),
        out_shape=jax.ShapeDtypeStruct((N, D), jnp.float32),
    )(x3, s, q, g, b)


def kernel(x, edge_index, batch, edge_attr, W_rel, b_rel, W_root, W_lin, b_lin, gamma, beta):
    agg = _sc_agg(x, edge_index[0], edge_index[1], edge_attr)
    x3, s, q = _post(agg, x, W_rel.T, W_root.T, b_rel.reshape(1, D),
                     W_lin.T, b_lin.reshape(1, D))
    return _bn(x3, s, q, gamma.reshape(1, D), beta.reshape(1, D))


# speedup vs baseline: 1.1971x; 1.1971x over previous
import jax
import jax.numpy as jnp
from jax import lax
from jax.experimental import pallas as pl
from jax.experimental.pallas import tpu as pltpu
from jax.experimental.pallas import tpu_sc as plsc

N = 10000
E = 320000
D = 128
NEG_SLOPE = 0.01
EPS = 1e-5

NUM_CORES = 2
NUM_SUBCORES = 16
NUM_TILES = NUM_CORES * NUM_SUBCORES
N_PAD = 10240
ROWS_PER_TILE = N_PAD // NUM_TILES

CHUNK = 10000
NCHUNK = E // CHUNK
VPC = 16
CHECK_VREGS = 25
CHECKS_PER_CHUNK = CHUNK // (CHECK_VREGS * VPC)
FIRE = 128
PCAP = 640


def _leaky(v):
    return jnp.where(v >= 0, v, NEG_SLOPE * v)


def _sc_agg_body(y_hbm, src_hbm, dst_hbm, ea_hbm, out_hbm,
                 srcc0, dstc0, eac0, srcc1, dstc1, eac1,
                 pend, psrc, pdl, pea, rows_v, acc, sem0, sem1, gsem):
    c = lax.axis_index("c")
    s = lax.axis_index("s")
    tid = c * NUM_SUBCORES + s
    tid_vec = jnp.full((VPC,), tid, jnp.int32)
    iota = lax.iota(jnp.int32, VPC)
    zeros16 = jnp.zeros((VPC,), jnp.float32)
    zeros16i = jnp.zeros((VPC,), jnp.int32)

    def zrow(r, carry):
        for i in range(D // VPC):
            acc[r, pl.ds(i * VPC, VPC)] = zeros16
        return carry
    lax.fori_loop(0, ROWS_PER_TILE, zrow, 0)
    for t in range(FIRE // VPC):
        psrc[pl.ds(t * VPC, VPC)] = zeros16i

    def prefetch(g, srcb, dstb, eab, sem):
        o = g * CHUNK
        pltpu.async_copy(src_hbm.at[pl.ds(o, CHUNK)], srcb.at[pl.ds(0, CHUNK)], sem)
        pltpu.async_copy(dst_hbm.at[pl.ds(o, CHUNK)], dstb.at[pl.ds(0, CHUNK)], sem)
        pltpu.async_copy(ea_hbm.at[pl.ds(o, CHUNK)], eab.at[pl.ds(0, CHUNK)], sem)

    def wait_prefetch(g, srcb, dstb, eab, sem):
        o = g * CHUNK
        pltpu.make_async_copy(src_hbm.at[pl.ds(o, CHUNK)], srcb.at[pl.ds(0, CHUNK)], sem).wait()
        pltpu.make_async_copy(dst_hbm.at[pl.ds(o, CHUNK)], dstb.at[pl.ds(0, CHUNK)], sem).wait()
        pltpu.make_async_copy(ea_hbm.at[pl.ds(o, CHUNK)], eab.at[pl.ds(0, CHUNK)], sem).wait()

    def make_scan_drain(srcb, dstb, eab):
        def append_vreg(base, np_vec):
            dvec = dstb[pl.ds(base, VPC)]
            owner = lax.shift_right_logical(
                lax.shift_right_logical(dvec, 6) * 52429, 18)
            m = owner == tid_vec
            m_i = jnp.where(m, 1, 0).astype(jnp.int32)
            pos = plsc.cumsum(m_i) - m_i
            cnt = plsc.all_reduce_population_count(m)
            plsc.store_scatter(pend, [np_vec + pos], iota + base, mask=m)
            return np_vec + cnt

        def fire(f, np_s):
            off = pl.multiple_of(f * FIRE, FIRE)
            cnt_v = zeros16i + (np_s - off)
            for t in range(FIRE // VPC):
                o = pl.multiple_of(off + t * VPC, VPC)
                valid = (iota + (t * VPC)) < cnt_v
                p16 = jnp.where(valid, pend[pl.ds(o, VPC)], 0)
                sv = plsc.load_gather(srcb, [p16])
                av = plsc.load_gather(eab, [p16])
                dv = plsc.load_gather(dstb, [p16])
                av = jnp.where(valid, av, 0.0)
                owner = lax.shift_right_logical(
                    lax.shift_right_logical(dv, 6) * 52429, 18)
                psrc[pl.ds(t * VPC, VPC)] = sv
                pea[pl.ds(t * VPC, VPC)] = av
                pdl[pl.ds(t * VPC, VPC)] = dv - owner * ROWS_PER_TILE
            pltpu.async_copy(y_hbm.at[psrc.at[pl.ds(0, FIRE)]], rows_v, gsem).wait()

            def consume(g, c2):
                for kk in range(VPC):
                    pidx = zeros16i + (g * VPC + kk)
                    ea_b = plsc.load_gather(pea, [pidx])
                    dl_b = plsc.load_gather(pdl, [pidx])
                    for i in range(D // VPC):
                        v = rows_v[g * VPC + kk, pl.ds(i * VPC, VPC)] * ea_b
                        plsc.addupdate_scatter(acc, [dl_b, iota + (i * VPC)], v)
                return c2
            ng = jnp.minimum(np_s - off, FIRE) + (VPC - 1)
            lax.fori_loop(0, ng // VPC, consume, 0)
            return np_s

        def scan_drain(np_vec):
            def app(u, np_vec3):
                for uu in range(5):
                    np_vec3 = append_vreg((u * 5 + uu) * VPC, np_vec3)
                return np_vec3
            np_vec = lax.fori_loop(0, CHUNK // (5 * VPC), app, np_vec)
            np_s = jnp.max(np_vec)
            nf = (np_s + FIRE - 1) // FIRE
            lax.fori_loop(0, nf, fire, np_s)
            return zeros16i
        return scan_drain

    scan_drain0 = make_scan_drain(srcc0, dstc0, eac0)
    scan_drain1 = make_scan_drain(srcc1, dstc1, eac1)

    prefetch(0, srcc0, dstc0, eac0, sem0)

    def pair(h, np_vec):
        g0 = h * 2
        wait_prefetch(g0, srcc0, dstc0, eac0, sem0)
        prefetch(g0 + 1, srcc1, dstc1, eac1, sem1)
        np_vec = scan_drain0(np_vec)
        wait_prefetch(g0 + 1, srcc1, dstc1, eac1, sem1)

        @pl.when(h < NCHUNK // 2 - 1)
        def _():
            prefetch(g0 + 2, srcc0, dstc0, eac0, sem0)
        return scan_drain1(np_vec)

    lax.fori_loop(0, NCHUNK // 2, pair, zeros16i)

    pltpu.sync_copy(acc, out_hbm.at[pl.ds(tid * ROWS_PER_TILE, ROWS_PER_TILE)])


_sc_agg = pl.kernel(
    _sc_agg_body,
    out_type=jax.ShapeDtypeStruct((N_PAD, D), jnp.float32),
    mesh=plsc.VectorSubcoreMesh(core_axis_name="c", subcore_axis_name="s",
                                num_cores=NUM_CORES,
                                num_subcores=NUM_SUBCORES),
    compiler_params=pltpu.CompilerParams(needs_layout_passes=False),
    scratch_types=[
        pltpu.VMEM((CHUNK,), jnp.int32),
        pltpu.VMEM((CHUNK,), jnp.int32),
        pltpu.VMEM((CHUNK,), jnp.float32),
        pltpu.VMEM((CHUNK,), jnp.int32),
        pltpu.VMEM((CHUNK,), jnp.int32),
        pltpu.VMEM((CHUNK,), jnp.float32),
        pltpu.VMEM((CHUNK + VPC,), jnp.int32),
        pltpu.VMEM((FIRE,), jnp.int32),
        pltpu.VMEM((FIRE,), jnp.int32),
        pltpu.VMEM((FIRE,), jnp.float32),
        pltpu.VMEM((FIRE, D), jnp.float32),
        pltpu.VMEM((ROWS_PER_TILE, D), jnp.float32),
        pltpu.SemaphoreType.DMA,
        pltpu.SemaphoreType.DMA,
        pltpu.SemaphoreType.DMA,
    ],
)


def _post_body(a_ref, x_ref, wr_ref, wt_ref, br_ref, wl_ref, bl_ref,
               x3_ref, s_ref, q_ref):
    i = pl.program_id(0)
    x1 = (jnp.dot(a_ref[...], wr_ref[...], preferred_element_type=jnp.float32)
          + jnp.dot(x_ref[...], wt_ref[...], preferred_element_type=jnp.float32)
          + br_ref[...])
    x2 = _leaky(x1)
    x3 = jnp.dot(x2, wl_ref[...], preferred_element_type=jnp.float32) + bl_ref[...]
    x3_ref[...] = x3

    @pl.when(i == 0)
    def _():
        s_ref[...] = jnp.zeros_like(s_ref)
        q_ref[...] = jnp.zeros_like(q_ref)

    s_ref[...] += jnp.sum(x3, axis=0, keepdims=True)
    q_ref[...] += jnp.sum(x3 * x3, axis=0, keepdims=True)


def _post(a, x, wrT, wtT, br, wlT, bl):
    blk = 1000
    grid = N // blk
    return pl.pallas_call(
        _post_body,
        grid=(grid,),
        in_specs=[
            pl.BlockSpec((blk, D), lambda i: (i, 0)),
            pl.BlockSpec((blk, D), lambda i: (i, 0)),
            pl.BlockSpec((D, D), lambda i: (0, 0)),
            pl.BlockSpec((D, D), lambda i: (0, 0)),
            pl.BlockSpec((1, D), lambda i: (0, 0)),
            pl.BlockSpec((D, D), lambda i: (0, 0)),
            pl.BlockSpec((1, D), lambda i: (0, 0)),
        ],
        out_specs=[
            pl.BlockSpec((blk, D), lambda i: (i, 0)),
            pl.BlockSpec((1, D), lambda i: (0, 0)),
            pl.BlockSpec((1, D), lambda i: (0, 0)),
        ],
        out_shape=[
            jax.ShapeDtypeStruct((N, D), jnp.float32),
            jax.ShapeDtypeStruct((1, D), jnp.float32),
            jax.ShapeDtypeStruct((1, D), jnp.float32),
        ],
    )(a, x, wrT, wtT, br, wlT, bl)


def _bn_body(x3_ref, s_ref, q_ref, g_ref, b_ref, o_ref):
    mean = s_ref[...] / N
    var = q_ref[...] / N - mean * mean
    scale = lax.rsqrt(var + EPS) * g_ref[...]
    x4 = (x3_ref[...] - mean) * scale + b_ref[...]
    o_ref[...] = _leaky(x4)


def _bn(x3, s, q, g, b):
    blk = 1000
    grid = N // blk
    return pl.pallas_call(
        _bn_body,
        grid=(grid,),
        in_specs=[
            pl.BlockSpec((blk, D), lambda i: (i, 0)),
            pl.BlockSpec((1, D), lambda i: (0, 0)),
            pl.BlockSpec((1, D), lambda i: (0, 0)),
            pl.BlockSpec((1, D), lambda i: (0, 0)),
            pl.BlockSpec((1, D), lambda i: (0, 0)),
        ],
        out_specs=pl.BlockSpec((blk, D), lambda i: (i, 0)),
        out_shape=jax.ShapeDtypeStruct((N, D), jnp.float32),
    )(x3, s, q, g, b)


def kernel(x, edge_index, batch, edge_attr, W_rel, b_rel, W_root, W_lin, b_lin, gamma, beta):
    agg = _sc_agg(x, edge_index[0], edge_index[1], edge_attr)
    x3, s, q = _post(agg, x, W_rel.T, W_root.T, b_rel.reshape(1, D),
                     W_lin.T, b_lin.reshape(1, D))
    return _bn(x3, s, q, gamma.reshape(1, D), beta.reshape(1, D))


# speedup vs baseline: 1.4215x; 1.1875x over previous
import jax
import jax.numpy as jnp
from jax import lax
from jax.experimental import pallas as pl
from jax.experimental.pallas import tpu as pltpu
from jax.experimental.pallas import tpu_sc as plsc

N = 10000
E = 320000
D = 128
NEG_SLOPE = 0.01
EPS = 1e-5

NUM_CORES = 2
NUM_SUBCORES = 16
NUM_TILES = NUM_CORES * NUM_SUBCORES
N_PAD = 10240
ROWS_PER_TILE = N_PAD // NUM_TILES

CHUNK = 10000
NCHUNK = E // CHUNK
VPC = 16
CHECK_VREGS = 25
CHECKS_PER_CHUNK = CHUNK // (CHECK_VREGS * VPC)
FIRE = 80
PCAP = 640


def _leaky(v):
    return jnp.where(v >= 0, v, NEG_SLOPE * v)


def _sc_agg_body(y_hbm, src_hbm, dst_hbm, ea_hbm, out_hbm,
                 srcc, dstc, eac, psrc, pdl, pea, rows_v, acc, sem):
    c = lax.axis_index("c")
    s = lax.axis_index("s")
    tid = c * NUM_SUBCORES + s
    tid_vec = jnp.full((VPC,), tid, jnp.int32)
    iota = lax.iota(jnp.int32, VPC)
    zeros16 = jnp.zeros((VPC,), jnp.float32)
    zeros16i = jnp.zeros((VPC,), jnp.int32)

    def zrow(r, carry):
        for i in range(D // VPC):
            acc[r, pl.ds(i * VPC, VPC)] = zeros16
        return carry
    lax.fori_loop(0, ROWS_PER_TILE, zrow, 0)

    def fire(f, carry):
        off = pl.multiple_of(f * FIRE, FIRE)
        pltpu.async_copy(y_hbm.at[psrc.at[pl.ds(off, FIRE)]], rows_v, sem).wait()

        def consume(k2, c2):
            for kk in range(2):
                k = k2 * 2 + kk
                pidx = jnp.full((VPC,), off + k, jnp.int32)
                ea_b = plsc.load_gather(pea, [pidx])
                dl_b = plsc.load_gather(pdl, [pidx])
                for i in range(D // VPC):
                    v = rows_v[k, pl.ds(i * VPC, VPC)] * ea_b
                    plsc.addupdate_scatter(acc, [dl_b, iota + (i * VPC)], v)
            return c2
        lax.fori_loop(0, FIRE // 2, consume, 0)
        return carry

    def append_vreg(base, np_vec, mask_all):
        dvec = dstc[pl.ds(base, VPC)]
        svec = srcc[pl.ds(base, VPC)]
        avec = eac[pl.ds(base, VPC)]
        owner = lax.shift_right_logical(
            lax.shift_right_logical(dvec, 6) * 52429, 18)
        if mask_all:
            m = jnp.ones((VPC,), jnp.bool_)
            pos = iota
            cnt = jnp.full((VPC,), VPC, jnp.int32)
        else:
            m = owner == tid_vec
            m_i = jnp.where(m, 1, 0).astype(jnp.int32)
            pos = plsc.cumsum(m_i) - m_i
            cnt = plsc.all_reduce_population_count(m)
        local = dvec - owner * ROWS_PER_TILE
        idx = np_vec + pos
        plsc.store_scatter(psrc, [idx], svec, mask=m)
        plsc.store_scatter(pea, [idx], avec, mask=m)
        plsc.store_scatter(pdl, [idx], local, mask=m)
        return np_vec + cnt

    def drain_fires(np_vec):
        np_s = jnp.max(np_vec)
        nf = np_s // FIRE
        lax.fori_loop(0, nf, fire, 0)
        sh = nf * FIRE
        for t in range(FIRE // VPC):
            o = pl.multiple_of(sh + t * VPC, VPC)
            psrc[pl.ds(t * VPC, VPC)] = psrc[pl.ds(o, VPC)]
            pea[pl.ds(t * VPC, VPC)] = pea[pl.ds(o, VPC)]
            pdl[pl.ds(t * VPC, VPC)] = pdl[pl.ds(o, VPC)]
        left = np_s - sh
        return jnp.full((VPC,), 1, jnp.int32) * left

    def chunk_body(g, np_vec):
        pltpu.sync_copy(src_hbm.at[pl.ds(g * CHUNK, CHUNK)], srcc)
        pltpu.sync_copy(dst_hbm.at[pl.ds(g * CHUNK, CHUNK)], dstc)
        pltpu.sync_copy(ea_hbm.at[pl.ds(g * CHUNK, CHUNK)], eac)

        def check_body(k, np_vec2):
            def app(u, np_vec3):
                for uu in range(5):
                    base = (k * CHECK_VREGS + u * 5 + uu) * VPC
                    np_vec3 = append_vreg(base, np_vec3, False)
                return np_vec3
            np_vec2 = lax.fori_loop(0, CHECK_VREGS // 5, app, np_vec2)
            return drain_fires(np_vec2)
        return lax.fori_loop(0, CHECKS_PER_CHUNK, check_body, np_vec)

    np_vec = lax.fori_loop(0, NCHUNK, chunk_body, zeros16i)

    def fake(u, np_vec2):
        srcc[pl.ds(u * VPC, VPC)] = zeros16i
        dstc[pl.ds(u * VPC, VPC)] = tid_vec * ROWS_PER_TILE
        eac[pl.ds(u * VPC, VPC)] = zeros16
        return append_vreg(u * VPC, np_vec2, True)
    np_vec = lax.fori_loop(0, FIRE // VPC, fake, np_vec)
    drain_fires(np_vec)

    pltpu.sync_copy(acc, out_hbm.at[pl.ds(tid * ROWS_PER_TILE, ROWS_PER_TILE)])


_sc_agg = pl.kernel(
    _sc_agg_body,
    out_type=jax.ShapeDtypeStruct((N_PAD, D), jnp.float32),
    mesh=plsc.VectorSubcoreMesh(core_axis_name="c", subcore_axis_name="s",
                                num_cores=NUM_CORES,
                                num_subcores=NUM_SUBCORES),
    compiler_params=pltpu.CompilerParams(needs_layout_passes=False),
    scratch_types=[
        pltpu.VMEM((CHUNK,), jnp.int32),
        pltpu.VMEM((CHUNK,), jnp.int32),
        pltpu.VMEM((CHUNK,), jnp.float32),
        pltpu.VMEM((PCAP,), jnp.int32),
        pltpu.VMEM((PCAP,), jnp.int32),
        pltpu.VMEM((PCAP,), jnp.float32),
        pltpu.VMEM((FIRE, D), jnp.float32),
        pltpu.VMEM((ROWS_PER_TILE, D), jnp.float32),
        pltpu.SemaphoreType.DMA,
    ],
)


def _post_body(a_ref, x_ref, wr_ref, wt_ref, br_ref, wl_ref, bl_ref,
               x3_ref, s_ref, q_ref):
    i = pl.program_id(0)
    x1 = (jnp.dot(a_ref[...], wr_ref[...], preferred_element_type=jnp.float32)
          + jnp.dot(x_ref[...], wt_ref[...], preferred_element_type=jnp.float32)
          + br_ref[...])
    x2 = _leaky(x1)
    x3 = jnp.dot(x2, wl_ref[...], preferred_element_type=jnp.float32) + bl_ref[...]
    x3_ref[...] = x3

    @pl.when(i == 0)
    def _():
        s_ref[...] = jnp.zeros_like(s_ref)
        q_ref[...] = jnp.zeros_like(q_ref)

    s_ref[...] += jnp.sum(x3, axis=0, keepdims=True)
    q_ref[...] += jnp.sum(x3 * x3, axis=0, keepdims=True)


def _post(a, x, wrT, wtT, br, wlT, bl):
    blk = 1000
    grid = N // blk
    return pl.pallas_call(
        _post_body,
        grid=(grid,),
        in_specs=[
            pl.BlockSpec((blk, D), lambda i: (i, 0)),
            pl.BlockSpec((blk, D), lambda i: (i, 0)),
            pl.BlockSpec((D, D), lambda i: (0, 0)),
            pl.BlockSpec((D, D), lambda i: (0, 0)),
            pl.BlockSpec((1, D), lambda i: (0, 0)),
            pl.BlockSpec((D, D), lambda i: (0, 0)),
            pl.BlockSpec((1, D), lambda i: (0, 0)),
        ],
        out_specs=[
            pl.BlockSpec((blk, D), lambda i: (i, 0)),
            pl.BlockSpec((1, D), lambda i: (0, 0)),
            pl.BlockSpec((1, D), lambda i: (0, 0)),
        ],
        out_shape=[
            jax.ShapeDtypeStruct((N, D), jnp.float32),
            jax.ShapeDtypeStruct((1, D), jnp.float32),
            jax.ShapeDtypeStruct((1, D), jnp.float32),
        ],
    )(a, x, wrT, wtT, br, wlT, bl)


def _bn_body(x3_ref, s_ref, q_ref, g_ref, b_ref, o_ref):
    mean = s_ref[...] / N
    var = q_ref[...] / N - mean * mean
    scale = lax.rsqrt(var + EPS) * g_ref[...]
    x4 = (x3_ref[...] - mean) * scale + b_ref[...]
    o_ref[...] = _leaky(x4)


def _bn(x3, s, q, g, b):
    blk = 1000
    grid = N // blk
    return pl.pallas_call(
        _bn_body,
        grid=(grid,),
        in_specs=[
            pl.BlockSpec((blk, D), lambda i: (i, 0)),
            pl.BlockSpec((1, D), lambda i: (0, 0)),
            pl.BlockSpec((1, D), lambda i: (0, 0)),
            pl.BlockSpec((1, D), lambda i: (0, 0)),
            pl.BlockSpec((1, D), lambda i: (0, 0)),
        ],
        out_specs=pl.BlockSpec((blk, D), lambda i: (i, 0)),
        out_shape=jax.ShapeDtypeStruct((N, D), jnp.float32),
    )(x3, s, q, g, b)


def kernel(x, edge_index, batch, edge_attr, W_rel, b_rel, W_root, W_lin, b_lin, gamma, beta):
    agg = _sc_agg(x, edge_index[0], edge_index[1], edge_attr)
    x3, s, q = _post(agg, x, W_rel.T, W_root.T, b_rel.reshape(1, D),
                     W_lin.T, b_lin.reshape(1, D))
    return _bn(x3, s, q, gamma.reshape(1, D), beta.reshape(1, D))


# speedup vs baseline: 1.5385x; 1.0823x over previous
import jax
import jax.numpy as jnp
from jax import lax
from jax.experimental import pallas as pl
from jax.experimental.pallas import tpu as pltpu
from jax.experimental.pallas import tpu_sc as plsc

N = 10000
E = 320000
D = 128
NEG_SLOPE = 0.01
EPS = 1e-5

NUM_CORES = 2
NUM_SUBCORES = 16
NUM_TILES = NUM_CORES * NUM_SUBCORES
N_PAD = 10240
ROWS_PER_TILE = N_PAD // NUM_TILES

CHUNK = 10000
NCHUNK = E // CHUNK
VPC = 16
CHECK_VREGS = 25
CHECKS_PER_CHUNK = CHUNK // (CHECK_VREGS * VPC)
FIRE = 80
PCAP = 640


def _leaky(v):
    return jnp.where(v >= 0, v, NEG_SLOPE * v)


def _sc_agg_body(y_hbm, src_hbm, dst_hbm, ea_hbm, out_hbm,
                 srcc, dstc, eac, srcc1, dstc1, eac1,
                 psrc, pdl, pea, rows_v, acc, sem0, sem1, gsem):
    c = lax.axis_index("c")
    s = lax.axis_index("s")
    tid = c * NUM_SUBCORES + s
    tid_vec = jnp.full((VPC,), tid, jnp.int32)
    iota = lax.iota(jnp.int32, VPC)
    zeros16 = jnp.zeros((VPC,), jnp.float32)
    zeros16i = jnp.zeros((VPC,), jnp.int32)

    def zrow(r, carry):
        for i in range(D // VPC):
            acc[r, pl.ds(i * VPC, VPC)] = zeros16
        return carry
    lax.fori_loop(0, ROWS_PER_TILE, zrow, 0)

    def fire(f, carry):
        off = pl.multiple_of(f * FIRE, FIRE)
        pltpu.async_copy(y_hbm.at[psrc.at[pl.ds(off, FIRE)]], rows_v, gsem).wait()

        def consume(k2, c2):
            for kk in range(2):
                k = k2 * 2 + kk
                pidx = jnp.full((VPC,), off + k, jnp.int32)
                ea_b = plsc.load_gather(pea, [pidx])
                dl_b = plsc.load_gather(pdl, [pidx])
                for i in range(D // VPC):
                    v = rows_v[k, pl.ds(i * VPC, VPC)] * ea_b
                    plsc.addupdate_scatter(acc, [dl_b, iota + (i * VPC)], v)
            return c2
        lax.fori_loop(0, FIRE // 2, consume, 0)
        return carry

    def append_vreg(base, np_vec, mask_all, sb, db, eb):
        dvec = db[pl.ds(base, VPC)]
        svec = sb[pl.ds(base, VPC)]
        avec = eb[pl.ds(base, VPC)]
        owner = lax.shift_right_logical(
            lax.shift_right_logical(dvec, 6) * 52429, 18)
        if mask_all:
            m = jnp.ones((VPC,), jnp.bool_)
            pos = iota
            cnt = jnp.full((VPC,), VPC, jnp.int32)
        else:
            m = owner == tid_vec
            m_i = jnp.where(m, 1, 0).astype(jnp.int32)
            pos = plsc.cumsum(m_i) - m_i
            cnt = plsc.all_reduce_population_count(m)
        local = dvec - owner * ROWS_PER_TILE
        idx = np_vec + pos
        plsc.store_scatter(psrc, [idx], svec, mask=m)
        plsc.store_scatter(pea, [idx], avec, mask=m)
        plsc.store_scatter(pdl, [idx], local, mask=m)
        return np_vec + cnt

    def drain_fires(np_vec):
        np_s = jnp.max(np_vec)
        nf = np_s // FIRE
        lax.fori_loop(0, nf, fire, 0)
        sh = nf * FIRE
        for t in range(FIRE // VPC):
            o = pl.multiple_of(sh + t * VPC, VPC)
            psrc[pl.ds(t * VPC, VPC)] = psrc[pl.ds(o, VPC)]
            pea[pl.ds(t * VPC, VPC)] = pea[pl.ds(o, VPC)]
            pdl[pl.ds(t * VPC, VPC)] = pdl[pl.ds(o, VPC)]
        left = np_s - sh
        return jnp.full((VPC,), 1, jnp.int32) * left

    def prefetch(g, sb, db, eb, sm):
        o = g * CHUNK
        pltpu.async_copy(src_hbm.at[pl.ds(o, CHUNK)], sb, sm)
        pltpu.async_copy(dst_hbm.at[pl.ds(o, CHUNK)], db, sm)
        pltpu.async_copy(ea_hbm.at[pl.ds(o, CHUNK)], eb, sm)

    def wait_prefetch(g, sb, db, eb, sm):
        o = g * CHUNK
        pltpu.make_async_copy(src_hbm.at[pl.ds(o, CHUNK)], sb, sm).wait()
        pltpu.make_async_copy(dst_hbm.at[pl.ds(o, CHUNK)], db, sm).wait()
        pltpu.make_async_copy(ea_hbm.at[pl.ds(o, CHUNK)], eb, sm).wait()

    def scan_chunk(sb, db, eb, np_vec):
        def check_body(k, np_vec2):
            def app(u, np_vec3):
                for uu in range(5):
                    base = (k * CHECK_VREGS + u * 5 + uu) * VPC
                    np_vec3 = append_vreg(base, np_vec3, False, sb, db, eb)
                return np_vec3
            np_vec2 = lax.fori_loop(0, CHECK_VREGS // 5, app, np_vec2)
            return drain_fires(np_vec2)
        return lax.fori_loop(0, CHECKS_PER_CHUNK, check_body, np_vec)

    prefetch(0, srcc, dstc, eac, sem0)

    def pair(h, np_vec):
        g0 = h * 2
        wait_prefetch(g0, srcc, dstc, eac, sem0)
        prefetch(g0 + 1, srcc1, dstc1, eac1, sem1)
        np_vec = scan_chunk(srcc, dstc, eac, np_vec)
        wait_prefetch(g0 + 1, srcc1, dstc1, eac1, sem1)

        @pl.when(h < NCHUNK // 2 - 1)
        def _():
            prefetch(g0 + 2, srcc, dstc, eac, sem0)
        return scan_chunk(srcc1, dstc1, eac1, np_vec)

    np_vec = lax.fori_loop(0, NCHUNK // 2, pair, zeros16i)

    def fake(u, np_vec2):
        srcc[pl.ds(u * VPC, VPC)] = zeros16i
        dstc[pl.ds(u * VPC, VPC)] = tid_vec * ROWS_PER_TILE
        eac[pl.ds(u * VPC, VPC)] = zeros16
        return append_vreg(u * VPC, np_vec2, True, srcc, dstc, eac)
    np_vec = lax.fori_loop(0, FIRE // VPC, fake, np_vec)
    drain_fires(np_vec)

    pltpu.sync_copy(acc, out_hbm.at[pl.ds(tid * ROWS_PER_TILE, ROWS_PER_TILE)])


_sc_agg = pl.kernel(
    _sc_agg_body,
    out_type=jax.ShapeDtypeStruct((N_PAD, D), jnp.float32),
    mesh=plsc.VectorSubcoreMesh(core_axis_name="c", subcore_axis_name="s",
                                num_cores=NUM_CORES,
                                num_subcores=NUM_SUBCORES),
    compiler_params=pltpu.CompilerParams(needs_layout_passes=False),
    scratch_types=[
        pltpu.VMEM((CHUNK,), jnp.int32),
        pltpu.VMEM((CHUNK,), jnp.int32),
        pltpu.VMEM((CHUNK,), jnp.float32),
        pltpu.VMEM((CHUNK,), jnp.int32),
        pltpu.VMEM((CHUNK,), jnp.int32),
        pltpu.VMEM((CHUNK,), jnp.float32),
        pltpu.VMEM((PCAP,), jnp.int32),
        pltpu.VMEM((PCAP,), jnp.int32),
        pltpu.VMEM((PCAP,), jnp.float32),
        pltpu.VMEM((FIRE, D), jnp.float32),
        pltpu.VMEM((ROWS_PER_TILE, D), jnp.float32),
        pltpu.SemaphoreType.DMA,
        pltpu.SemaphoreType.DMA,
        pltpu.SemaphoreType.DMA,
    ],
)


def _post_body(a_ref, x_ref, wr_ref, wt_ref, br_ref, wl_ref, bl_ref,
               x3_ref, s_ref, q_ref):
    i = pl.program_id(0)
    x1 = (jnp.dot(a_ref[...], wr_ref[...], preferred_element_type=jnp.float32)
          + jnp.dot(x_ref[...], wt_ref[...], preferred_element_type=jnp.float32)
          + br_ref[...])
    x2 = _leaky(x1)
    x3 = jnp.dot(x2, wl_ref[...], preferred_element_type=jnp.float32) + bl_ref[...]
    x3_ref[...] = x3

    @pl.when(i == 0)
    def _():
        s_ref[...] = jnp.zeros_like(s_ref)
        q_ref[...] = jnp.zeros_like(q_ref)

    s_ref[...] += jnp.sum(x3, axis=0, keepdims=True)
    q_ref[...] += jnp.sum(x3 * x3, axis=0, keepdims=True)


def _post(a, x, wrT, wtT, br, wlT, bl):
    blk = 1000
    grid = N // blk
    return pl.pallas_call(
        _post_body,
        grid=(grid,),
        in_specs=[
            pl.BlockSpec((blk, D), lambda i: (i, 0)),
            pl.BlockSpec((blk, D), lambda i: (i, 0)),
            pl.BlockSpec((D, D), lambda i: (0, 0)),
            pl.BlockSpec((D, D), lambda i: (0, 0)),
            pl.BlockSpec((1, D), lambda i: (0, 0)),
            pl.BlockSpec((D, D), lambda i: (0, 0)),
            pl.BlockSpec((1, D), lambda i: (0, 0)),
        ],
        out_specs=[
            pl.BlockSpec((blk, D), lambda i: (i, 0)),
            pl.BlockSpec((1, D), lambda i: (0, 0)),
            pl.BlockSpec((1, D), lambda i: (0, 0)),
        ],
        out_shape=[
            jax.ShapeDtypeStruct((N, D), jnp.float32),
            jax.ShapeDtypeStruct((1, D), jnp.float32),
            jax.ShapeDtypeStruct((1, D), jnp.float32),
        ],
    )(a, x, wrT, wtT, br, wlT, bl)


def _bn_body(x3_ref, s_ref, q_ref, g_ref, b_ref, o_ref):
    mean = s_ref[...] / N
    var = q_ref[...] / N - mean * mean
    scale = lax.rsqrt(var + EPS) * g_ref[...]
    x4 = (x3_ref[...] - mean) * scale + b_ref[...]
    o_ref[...] = _leaky(x4)


def _bn(x3, s, q, g, b):
    blk = 1000
    grid = N // blk
    return pl.pallas_call(
        _bn_body,
        grid=(grid,),
        in_specs=[
            pl.BlockSpec((blk, D), lambda i: (i, 0)),
            pl.BlockSpec((1, D), lambda i: (0, 0)),
            pl.BlockSpec((1, D), lambda i: (0, 0)),
            pl.BlockSpec((1, D), lambda i: (0, 0)),
            pl.BlockSpec((1, D), lambda i: (0, 0)),
        ],
        out_specs=pl.BlockSpec((blk, D), lambda i: (i, 0)),
        out_shape=jax.ShapeDtypeStruct((N, D), jnp.float32),
    )(x3, s, q, g, b)


def kernel(x, edge_index, batch, edge_attr, W_rel, b_rel, W_root, W_lin, b_lin, gamma, beta):
    agg = _sc_agg(x, edge_index[0], edge_index[1], edge_attr)
    x3, s, q = _post(agg, x, W_rel.T, W_root.T, b_rel.reshape(1, D),
                     W_lin.T, b_lin.reshape(1, D))
    return _bn(x3, s, q, gamma.reshape(1, D), beta.reshape(1, D))
